# baseline (speedup 1.0000x reference)
import functools

import jax
import jax.numpy as jnp
from jax import lax
from jax.experimental import pallas as pl
from jax.experimental.pallas import tpu as pltpu
from jax.experimental.pallas import tpu_sc as plsc

N = 50000
N_PAD = 50176
E = 800000
NW = 32
EPT = E // NW
NB = 196
NBLK = 256
CHD = 1000
CHS = 1024
NCHS = 25
E_IN = 800768
SCRATCH = 801376
CAP = SCRATCH + NW * CHS

_MESH = plsc.VectorSubcoreMesh(core_axis_name="c", subcore_axis_name="s")
_NOTC = pltpu.CompilerParams(use_tc_tiling_on_sc=False)


def _wid():
    return lax.axis_index("s") * 2 + lax.axis_index("c")


def _lane():
    return lax.iota(jnp.int32, 16)


@functools.partial(
    pl.kernel,
    out_type=(jax.ShapeDtypeStruct((NW, N_PAD), jnp.float32),
              jax.ShapeDtypeStruct((NW * 256,), jnp.float32)),
    mesh=_MESH,
    compiler_params=_NOTC,
    scratch_types=[pltpu.VMEM((N_PAD + 16,), jnp.float32),
                   pltpu.VMEM((272,), jnp.float32),
                   pltpu.VMEM((1008,), jnp.int32),
                   pltpu.VMEM((1008,), jnp.int32),
                   pltpu.VMEM((1008,), jnp.float32)],
)
def _k_deghist(src_h, dst_h, ew_h, pdeg_h, hist_h, deg_v, hist_v, sv, dv, ewv):
    wid = _wid()
    lane = _lane()

    def zd(i, _):
        deg_v[pl.ds(i * 16, 16)] = jnp.zeros((16,), jnp.float32)
        return 0
    lax.fori_loop(0, (N_PAD + 16) // 16, zd, 0)

    def zh(i, _):
        hist_v[pl.ds(i * 16, 16)] = jnp.zeros((16,), jnp.float32)
        return 0
    lax.fori_loop(0, 17, zh, 0)

    base = wid * EPT
    one0 = jnp.where(lane == 0, 1.0, 0.0)

    def chunk(c, _):
        eo = base + c * CHD
        pltpu.sync_copy(src_h.at[pl.ds(eo, CHD)], sv.at[pl.ds(0, CHD)])
        pltpu.sync_copy(dst_h.at[pl.ds(eo, CHD)], dv.at[pl.ds(0, CHD)])
        pltpu.sync_copy(ew_h.at[pl.ds(eo, CHD)], ewv.at[pl.ds(0, CHD)])

        def grp(g, _):
            sl = pl.ds(g * 16, 16)
            s16 = sv[sl]
            e16 = ewv[sl]
            b16 = lax.shift_right_logical(dv[sl], 8)
            for l in range(16):
                plsc.addupdate(deg_v.at[pl.ds(s16[l], 16)],
                               jnp.where(lane == 0, e16[l], 0.0))
                plsc.addupdate(hist_v.at[pl.ds(b16[l], 16)], one0)
            return 0
        lax.fori_loop(0, 62, grp, 0)
        s16 = sv[pl.ds(992, 16)]
        e16 = ewv[pl.ds(992, 16)]
        b16 = lax.shift_right_logical(dv[pl.ds(992, 16)], 8)
        for l in range(8):
            plsc.addupdate(deg_v.at[pl.ds(s16[l], 16)],
                           jnp.where(lane == 0, e16[l], 0.0))
            plsc.addupdate(hist_v.at[pl.ds(b16[l], 16)], one0)
        return 0
    lax.fori_loop(0, EPT // CHD, chunk, 0)

    pltpu.sync_copy(deg_v.at[pl.ds(0, N_PAD)], pdeg_h.at[wid])
    pltpu.sync_copy(hist_v.at[pl.ds(0, 256)], hist_h.at[pl.ds(wid * 256, 256)])


COLS = N_PAD // NW


@functools.partial(
    pl.kernel,
    out_type=jax.ShapeDtypeStruct((N_PAD,), jnp.float32),
    mesh=_MESH,
    compiler_params=_NOTC,
    scratch_types=[pltpu.VMEM((COLS,), jnp.float32),
                   pltpu.VMEM((COLS,), jnp.float32)],
)
def _k_dinv(pdeg_h, dinv_h, acc_v, buf_v):
    wid = _wid()
    co = wid * COLS

    def z(i, _):
        acc_v[pl.ds(i * 16, 16)] = jnp.zeros((16,), jnp.float32)
        return 0
    lax.fori_loop(0, COLS // 16, z, 0)

    def row(t, _):
        pltpu.sync_copy(pdeg_h.at[t, pl.ds(co, COLS)], buf_v)

        def add(g, _):
            sl = pl.ds(g * 16, 16)
            acc_v[sl] = acc_v[sl] + buf_v[sl]
            return 0
        lax.fori_loop(0, COLS // 16, add, 0)
        return 0
    lax.fori_loop(0, NW, row, 0)

    def rsq(g, _):
        sl = pl.ds(g * 16, 16)
        d = acc_v[sl]
        ds = jnp.where(d > 0, d, 1.0)
        s = jnp.maximum(ds, 1.0)
        for _it in range(18):
            s = 0.5 * (s + ds / s)
        acc_v[sl] = jnp.where(d > 0, 1.0 / s, 0.0)
        return 0
    lax.fori_loop(0, COLS // 16, rsq, 0)
    pltpu.sync_copy(acc_v, dinv_h.at[pl.ds(co, COLS)])


@functools.partial(
    pl.kernel,
    out_type=(jax.ShapeDtypeStruct((CAP,), jnp.int32),
              jax.ShapeDtypeStruct((CAP,), jnp.int32),
              jax.ShapeDtypeStruct((CAP,), jnp.float32),
              jax.ShapeDtypeStruct((256,), jnp.int32),
              jax.ShapeDtypeStruct((256,), jnp.int32)),
    mesh=_MESH,
    compiler_params=_NOTC,
    scratch_types=[pltpu.VMEM((NW * 256,), jnp.float32),
                   pltpu.SMEM((256,), jnp.int32),
                   pltpu.VMEM((256,), jnp.int32),
                   pltpu.VMEM((256,), jnp.int32),
                   pltpu.VMEM((CHS,), jnp.int32),
                   pltpu.VMEM((CHS,), jnp.int32),
                   pltpu.VMEM((CHS,), jnp.float32),
                   pltpu.VMEM((CHS,), jnp.float32),
                   pltpu.VMEM((CHS,), jnp.float32),
                   pltpu.VMEM((CHS,), jnp.float32),
                   pltpu.VMEM((8, 128), jnp.int32),
                   pltpu.SemaphoreType.DMA],
)
def _k_sort(src_h, dst_h, ew_h, dinv_h, hist_h,
            srcs_h, dsts_h, ws_h, offs_h, cnts_h,
            hist_v, ctr, offs_v, cnts_v, sv, dv, ewv, wv, dsv, ddv, pos2d, sem):
    wid = _wid()
    lane = _lane()
    pltpu.sync_copy(hist_h, hist_v)

    carry = jnp.int32(0)
    for grp in range(16):
        sl = pl.ds(grp * 16, 16)

        def tt(t, s):
            return s + hist_v[pl.ds(t * 256 + grp * 16, 16)]
        tot16 = lax.fori_loop(0, NW, tt, jnp.zeros((16,), jnp.float32))
        tot16i = tot16.astype(jnp.int32)

        def mt(t, s):
            return s + hist_v[pl.ds(t * 256 + grp * 16, 16)]
        my16 = lax.fori_loop(0, wid, mt, jnp.zeros((16,), jnp.float32))
        my16i = my16.astype(jnp.int32)

        pad16 = jnp.bitwise_and(tot16i + 7, jnp.int32(-8))
        spad16 = jnp.zeros((16,), jnp.int32)
        for l in range(16):
            spad16 = jnp.where(lane == l, carry, spad16)
            carry = carry + pad16[l]
        offs_v[sl] = spad16
        cnts_v[sl] = tot16i
        ini16 = spad16 + my16i
        for l in range(16):
            ctr[grp * 16 + l] = ini16[l]

    @pl.when(wid == 0)
    def _():
        pltpu.sync_copy(offs_v, offs_h)
        pltpu.sync_copy(cnts_v, cnts_h)

    base = wid * EPT

    def chunk(c, _):
        eo = base + c * CHS
        pltpu.sync_copy(src_h.at[pl.ds(eo, CHS)], sv)
        pltpu.sync_copy(dst_h.at[pl.ds(eo, CHS)], dv)
        pltpu.sync_copy(ew_h.at[pl.ds(eo, CHS)], ewv)
        gcp = []
        for j in range(8):
            sl = pl.ds(j * 128, 128)
            gcp.append(pltpu.async_copy(dinv_h.at[sv.at[sl]], dsv.at[sl], sem))
            gcp.append(pltpu.async_copy(dinv_h.at[dv.at[sl]], ddv.at[sl], sem))
        for cp in gcp:
            cp.wait()

        def vec(g, _):
            sl = pl.ds(g * 16, 16)
            wv[sl] = -(dsv[sl] * ewv[sl] * ddv[sl])
            return 0
        lax.fori_loop(0, CHS // 16, vec, 0)

        rem = jnp.minimum(EPT - c * CHS, CHS)

        def place(g, _):
            b16 = lax.shift_right_logical(dv[pl.ds(g * 16, 16)], 8)
            pvec = jnp.zeros((16,), jnp.int32)
            for l in range(16):
                bb = b16[l]
                i_abs = g * 16 + l
                is_real = i_abs < rem
                p_real = ctr[bb]
                p = jnp.where(is_real, p_real, SCRATCH + wid * CHS + i_abs)
                ctr[bb] = jnp.where(is_real, p_real + 1, p_real)
                pvec = jnp.where(lane == l, p, pvec)
            r = lax.shift_right_logical(g, 3)
            q = jnp.bitwise_and(g, 7) * 16
            pos2d[r, pl.ds(q, 16)] = pvec
            return 0
        lax.fori_loop(0, CHS // 16, place, 0)

        scp = []
        for j in range(8):
            idx = pos2d.at[j]
            sl = pl.ds(j * 128, 128)
            scp.append(pltpu.async_copy(sv.at[sl], srcs_h.at[idx], sem))
            scp.append(pltpu.async_copy(dv.at[sl], dsts_h.at[idx], sem))
            scp.append(pltpu.async_copy(wv.at[sl], ws_h.at[idx], sem))
        for cp in scp:
            cp.wait()
        return 0
    lax.fori_loop(0, NCHS, chunk, 0)


def _make_prop(D, first):
    GPE = D // 16
    FLATB = NBLK * GPE
    G = 64 if D == 256 else 128
    GSH = 6 if D == 256 else 7

    def body(*args):
        if first:
            (x_h, srcs_h, dsts_h, ws_h, offs_h, cnts_h, y_h,
             acc, rows2, xp, sidx2, wvv2, dvv2, offs_v, cnts_v, offs_sm,
             cnts_sm, sem_e, sem_g) = args
            xprev_h = None
        else:
            (x_h, xprev_h, srcs_h, dsts_h, ws_h, offs_h, cnts_h, y_h,
             acc, rows2, xp, sidx2, wvv2, dvv2, offs_v, cnts_v, offs_sm,
             cnts_sm, sem_e, sem_g) = args
        wid = _wid()
        lane = _lane()
        pltpu.sync_copy(offs_h, offs_v)
        pltpu.sync_copy(cnts_h, cnts_v)
        for sg in range(16):
            o16 = offs_v[pl.ds(sg * 16, 16)]
            c16 = cnts_v[pl.ds(sg * 16, 16)]
            for l in range(16):
                offs_sm[sg * 16 + l] = o16[l]
                cnts_sm[sg * 16 + l] = c16[l]

        def bucket(j, _):
            b = wid + NW * j

            @pl.when(b < NB)
            def _():
                nbase = b * NBLK
                fbase = b * FLATB

                def z(i, _):
                    acc[i] = jnp.zeros((16,), jnp.float32)
                    return 0
                lax.fori_loop(0, FLATB, z, 0)

                off0 = pl.multiple_of(offs_sm[b], 8)
                cnt = cnts_sm[b]
                nch = lax.shift_right_logical(cnt + (G - 1), GSH)

                def start_edata(c, slot):
                    eo = off0 + c * G
                    pltpu.async_copy(srcs_h.at[pl.ds(eo, G)], sidx2.at[slot],
                                     sem_e)
                    pltpu.async_copy(ws_h.at[pl.ds(eo, G)], wvv2.at[slot],
                                     sem_e)
                    pltpu.async_copy(dsts_h.at[pl.ds(eo, G)], dvv2.at[slot],
                                     sem_e)

                def wait_edata(slot):
                    pltpu.make_async_copy(srcs_h.at[pl.ds(0, G)],
                                          sidx2.at[slot], sem_e).wait()
                    pltpu.make_async_copy(ws_h.at[pl.ds(0, G)],
                                          wvv2.at[slot], sem_e).wait()
                    pltpu.make_async_copy(dsts_h.at[pl.ds(0, G)],
                                          dvv2.at[slot], sem_e).wait()

                def start_gather(slot):
                    def cl(g, _):
                        sl = pl.ds(g * 16, 16)
                        sidx2[slot, sl] = jnp.minimum(
                            jnp.maximum(sidx2[slot, sl], 0), N_PAD - 1)
                        return 0
                    lax.fori_loop(0, G // 16, cl, 0)
                    pltpu.async_copy(x_h.at[sidx2.at[slot]],
                                     rows2.at[pl.ds(slot * G, G)], sem_g)

                def wait_gather(slot):
                    pltpu.make_async_copy(x_h.at[pl.ds(0, G)],
                                          rows2.at[pl.ds(slot * G, G)],
                                          sem_g).wait()

                @pl.when(nch > 0)
                def _():
                    start_edata(0, 0)
                    wait_edata(0)
                    start_gather(0)

                    @pl.when(nch > 1)
                    def _():
                        start_edata(1, 1)

                def chunk(c, _):
                    slot = jnp.bitwise_and(c, 1)
                    wait_gather(slot)

                    @pl.when(c + 1 < nch)
                    def _():
                        wait_edata(1 - slot)
                        start_gather(1 - slot)

                    rem = cnt - c * G

                    @plsc.parallel_loop(0, G // 16, unroll=2)
                    def group(g):
                        sl = pl.ds(g * 16, 16)
                        w16 = jnp.where(lane < rem - g * 16,
                                        wvv2[slot, sl], 0.0)
                        d16 = jnp.minimum(
                            jnp.maximum(dvv2[slot, sl] - nbase, 0),
                            NBLK - 1) * GPE
                        rbase = slot * G + g * 16
                        for l in range(16):
                            fr = d16[l]
                            wi = w16[l]
                            for gg in range(GPE):
                                plsc.addupdate(
                                    acc.at[fr + gg],
                                    wi * rows2[rbase + l, pl.ds(gg * 16, 16)])

                    @pl.when(c + 2 < nch)
                    def _():
                        start_edata(c + 2, slot)
                    return 0
                lax.fori_loop(0, nch, chunk, 0)

                if first:
                    pltpu.sync_copy(acc, y_h.at[pl.ds(fbase, FLATB)])
                else:
                    def comb(q, _):
                        pltpu.sync_copy(
                            xprev_h.at[pl.ds(fbase + q * 512, 512)], xp)

                        def cv(v, _):
                            a = acc[q * 512 + v]
                            acc[q * 512 + v] = 2.0 * a - xp[v]
                            return 0
                        lax.fori_loop(0, 512, cv, 0)
                        return 0
                    lax.fori_loop(0, FLATB // 512, comb, 0)
                    pltpu.sync_copy(acc, y_h.at[pl.ds(fbase, FLATB)])
            return 0
        lax.fori_loop(0, (NB + NW - 1) // NW, bucket, 0)

    return pl.kernel(
        body,
        out_type=jax.ShapeDtypeStruct((N_PAD * GPE, 16), jnp.float32),
        mesh=_MESH,
        compiler_params=_NOTC,
        scratch_types=[pltpu.VMEM((FLATB, 16), jnp.float32),
                       pltpu.VMEM((2 * G, D), jnp.float32),
                       pltpu.VMEM((512, 16), jnp.float32),
                       pltpu.VMEM((2, G), jnp.int32),
                       pltpu.VMEM((2, G), jnp.float32),
                       pltpu.VMEM((2, G), jnp.int32),
                       pltpu.VMEM((256,), jnp.int32),
                       pltpu.VMEM((256,), jnp.int32),
                       pltpu.SMEM((256,), jnp.int32),
                       pltpu.SMEM((256,), jnp.int32),
                       pltpu.SemaphoreType.DMA,
                       pltpu.SemaphoreType.DMA],
    )


_prop64_first = _make_prop(64, True)
_prop64_rec = _make_prop(64, False)
_prop256_first = _make_prop(256, True)
_prop256_rec = _make_prop(256, False)


def _mm_body(relu):
    def body(x0, x1, x2, x3, x4, w_ref, b_ref, o_ref):
        W = w_ref[...]
        acc = jnp.dot(x0[...], W[0], preferred_element_type=jnp.float32)
        acc = acc + jnp.dot(x1[...], W[1], preferred_element_type=jnp.float32)
        acc = acc + jnp.dot(x2[...], W[2], preferred_element_type=jnp.float32)
        acc = acc + jnp.dot(x3[...], W[3], preferred_element_type=jnp.float32)
        acc = acc + jnp.dot(x4[...], W[4], preferred_element_type=jnp.float32)
        acc = acc + b_ref[...]
        if relu:
            acc = jnp.maximum(acc, 0.0)
        o_ref[...] = acc
    return body


def _mm(xs, Wt, b, relu):
    BM = 256
    DI = Wt.shape[1]
    DO = Wt.shape[2]
    in_specs = ([pl.BlockSpec((BM, DI), lambda i: (i, 0)) for _ in range(5)]
                + [pl.BlockSpec((5, DI, DO), lambda i: (0, 0, 0)),
                   pl.BlockSpec((1, DO), lambda i: (0, 0))])
    return pl.pallas_call(
        _mm_body(relu),
        grid=(N_PAD // BM,),
        in_specs=in_specs,
        out_specs=pl.BlockSpec((BM, DO), lambda i: (i, 0)),
        out_shape=jax.ShapeDtypeStruct((N_PAD, DO), jnp.float32),
    )(*xs, Wt, b.reshape(1, -1))


def kernel(x, edge_index, train_edge_weight, W1, b1, W2, b2):
    src = edge_index[0]
    dst = edge_index[1]
    srcp = jnp.pad(src, (0, E_IN - E))
    dstp = jnp.pad(dst, (0, E_IN - E))
    ewp = jnp.pad(train_edge_weight, (0, E_IN - E))
    xp = jnp.pad(x, ((0, N_PAD - N), (0, 0)))

    pdeg, hist = _k_deghist(srcp, dstp, ewp)
    dinv = _k_dinv(pdeg)
    es = _k_sort(srcp, dstp, ewp, dinv, hist)

    def flat(a, d):
        return a.reshape(N_PAD * (d // 16), 16)

    def unflat(a, d):
        return a.reshape(N_PAD, d)

    T0 = xp
    T0f = flat(T0, 64)
    T1f = _prop64_first(T0, *es)
    T1 = unflat(T1f, 64)
    T2f = _prop64_rec(T1, T0f, *es)
    T2 = unflat(T2f, 64)
    T3f = _prop64_rec(T2, T1f, *es)
    T3 = unflat(T3f, 64)
    T4f = _prop64_rec(T3, T2f, *es)
    T4 = unflat(T4f, 64)
    h = _mm([T0, T1, T2, T3, T4], W1, b1, relu=True)

    S0 = h
    S0f = flat(S0, 256)
    S1f = _prop256_first(S0, *es)
    S1 = unflat(S1f, 256)
    S2f = _prop256_rec(S1, S0f, *es)
    S2 = unflat(S2f, 256)
    S3f = _prop256_rec(S2, S1f, *es)
    S3 = unflat(S3f, 256)
    S4f = _prop256_rec(S3, S2f, *es)
    S4 = unflat(S4f, 256)
    out = _mm([S0, S1, S2, S3, S4], W2, b2, relu=False)

    return out[:N]


# speedup vs baseline: 1.0697x; 1.0697x over previous
import functools

import jax
import jax.numpy as jnp
from jax import lax
from jax.experimental import pallas as pl
from jax.experimental.pallas import tpu as pltpu
from jax.experimental.pallas import tpu_sc as plsc

N = 50000
N_PAD = 50176
E = 800000
NW = 32
EPT = E // NW
NB = 196
NBLK = 256
CHD = 1000
CHS = 1024
NCHS = 25
E_IN = 800768
SCRATCH = 801376
CAP = SCRATCH + NW * CHS

_MESH = plsc.VectorSubcoreMesh(core_axis_name="c", subcore_axis_name="s")
_NOTC = pltpu.CompilerParams(use_tc_tiling_on_sc=False)


def _wid():
    return lax.axis_index("s") * 2 + lax.axis_index("c")


def _lane():
    return lax.iota(jnp.int32, 16)


@functools.partial(
    pl.kernel,
    out_type=(jax.ShapeDtypeStruct((NW, N_PAD), jnp.float32),
              jax.ShapeDtypeStruct((NW * 256,), jnp.float32)),
    mesh=_MESH,
    compiler_params=_NOTC,
    scratch_types=[pltpu.VMEM((N_PAD + 16,), jnp.float32),
                   pltpu.VMEM((272,), jnp.float32),
                   pltpu.VMEM((1008,), jnp.int32),
                   pltpu.VMEM((1008,), jnp.int32),
                   pltpu.VMEM((1008,), jnp.float32)],
)
def _k_deghist(src_h, dst_h, ew_h, pdeg_h, hist_h, deg_v, hist_v, sv, dv, ewv):
    wid = _wid()
    lane = _lane()

    def zd(i, _):
        deg_v[pl.ds(i * 16, 16)] = jnp.zeros((16,), jnp.float32)
        return 0
    lax.fori_loop(0, (N_PAD + 16) // 16, zd, 0)

    def zh(i, _):
        hist_v[pl.ds(i * 16, 16)] = jnp.zeros((16,), jnp.float32)
        return 0
    lax.fori_loop(0, 17, zh, 0)

    base = wid * EPT
    one0 = jnp.where(lane == 0, 1.0, 0.0)

    def chunk(c, _):
        eo = base + c * CHD
        pltpu.sync_copy(src_h.at[pl.ds(eo, CHD)], sv.at[pl.ds(0, CHD)])
        pltpu.sync_copy(dst_h.at[pl.ds(eo, CHD)], dv.at[pl.ds(0, CHD)])
        pltpu.sync_copy(ew_h.at[pl.ds(eo, CHD)], ewv.at[pl.ds(0, CHD)])

        def grp(g, _):
            sl = pl.ds(g * 16, 16)
            s16 = sv[sl]
            e16 = ewv[sl]
            b16 = lax.shift_right_logical(dv[sl], 8)
            for l in range(16):
                plsc.addupdate(deg_v.at[pl.ds(s16[l], 16)],
                               jnp.where(lane == 0, e16[l], 0.0))
                plsc.addupdate(hist_v.at[pl.ds(b16[l], 16)], one0)
            return 0
        lax.fori_loop(0, 62, grp, 0)
        s16 = sv[pl.ds(992, 16)]
        e16 = ewv[pl.ds(992, 16)]
        b16 = lax.shift_right_logical(dv[pl.ds(992, 16)], 8)
        for l in range(8):
            plsc.addupdate(deg_v.at[pl.ds(s16[l], 16)],
                           jnp.where(lane == 0, e16[l], 0.0))
            plsc.addupdate(hist_v.at[pl.ds(b16[l], 16)], one0)
        return 0
    lax.fori_loop(0, EPT // CHD, chunk, 0)

    pltpu.sync_copy(deg_v.at[pl.ds(0, N_PAD)], pdeg_h.at[wid])
    pltpu.sync_copy(hist_v.at[pl.ds(0, 256)], hist_h.at[pl.ds(wid * 256, 256)])


COLS = N_PAD // NW


@functools.partial(
    pl.kernel,
    out_type=jax.ShapeDtypeStruct((N_PAD,), jnp.float32),
    mesh=_MESH,
    compiler_params=_NOTC,
    scratch_types=[pltpu.VMEM((COLS,), jnp.float32),
                   pltpu.VMEM((COLS,), jnp.float32)],
)
def _k_dinv(pdeg_h, dinv_h, acc_v, buf_v):
    wid = _wid()
    co = wid * COLS

    def z(i, _):
        acc_v[pl.ds(i * 16, 16)] = jnp.zeros((16,), jnp.float32)
        return 0
    lax.fori_loop(0, COLS // 16, z, 0)

    def row(t, _):
        pltpu.sync_copy(pdeg_h.at[t, pl.ds(co, COLS)], buf_v)

        def add(g, _):
            sl = pl.ds(g * 16, 16)
            acc_v[sl] = acc_v[sl] + buf_v[sl]
            return 0
        lax.fori_loop(0, COLS // 16, add, 0)
        return 0
    lax.fori_loop(0, NW, row, 0)

    def rsq(g, _):
        sl = pl.ds(g * 16, 16)
        d = acc_v[sl]
        ds = jnp.where(d > 0, d, 1.0)
        s = jnp.maximum(ds, 1.0)
        for _it in range(18):
            s = 0.5 * (s + ds / s)
        acc_v[sl] = jnp.where(d > 0, 1.0 / s, 0.0)
        return 0
    lax.fori_loop(0, COLS // 16, rsq, 0)
    pltpu.sync_copy(acc_v, dinv_h.at[pl.ds(co, COLS)])


@functools.partial(
    pl.kernel,
    out_type=(jax.ShapeDtypeStruct((CAP,), jnp.int32),
              jax.ShapeDtypeStruct((CAP,), jnp.int32),
              jax.ShapeDtypeStruct((CAP,), jnp.float32),
              jax.ShapeDtypeStruct((256,), jnp.int32),
              jax.ShapeDtypeStruct((256,), jnp.int32)),
    mesh=_MESH,
    compiler_params=_NOTC,
    scratch_types=[pltpu.VMEM((NW * 256,), jnp.float32),
                   pltpu.SMEM((256,), jnp.int32),
                   pltpu.VMEM((256,), jnp.int32),
                   pltpu.VMEM((256,), jnp.int32),
                   pltpu.VMEM((CHS,), jnp.int32),
                   pltpu.VMEM((CHS,), jnp.int32),
                   pltpu.VMEM((CHS,), jnp.float32),
                   pltpu.VMEM((CHS,), jnp.float32),
                   pltpu.VMEM((CHS,), jnp.float32),
                   pltpu.VMEM((CHS,), jnp.float32),
                   pltpu.VMEM((8, 128), jnp.int32),
                   pltpu.SemaphoreType.DMA],
)
def _k_sort(src_h, dst_h, ew_h, dinv_h, hist_h,
            srcs_h, dsts_h, ws_h, offs_h, cnts_h,
            hist_v, ctr, offs_v, cnts_v, sv, dv, ewv, wv, dsv, ddv, pos2d, sem):
    wid = _wid()
    lane = _lane()
    pltpu.sync_copy(hist_h, hist_v)

    carry = jnp.int32(0)
    for grp in range(16):
        sl = pl.ds(grp * 16, 16)

        def tt(t, s):
            return s + hist_v[pl.ds(t * 256 + grp * 16, 16)]
        tot16 = lax.fori_loop(0, NW, tt, jnp.zeros((16,), jnp.float32))
        tot16i = tot16.astype(jnp.int32)

        def mt(t, s):
            return s + hist_v[pl.ds(t * 256 + grp * 16, 16)]
        my16 = lax.fori_loop(0, wid, mt, jnp.zeros((16,), jnp.float32))
        my16i = my16.astype(jnp.int32)

        pad16 = jnp.bitwise_and(tot16i + 7, jnp.int32(-8))
        spad16 = jnp.zeros((16,), jnp.int32)
        for l in range(16):
            spad16 = jnp.where(lane == l, carry, spad16)
            carry = carry + pad16[l]
        offs_v[sl] = spad16
        cnts_v[sl] = tot16i
        ini16 = spad16 + my16i
        for l in range(16):
            ctr[grp * 16 + l] = ini16[l]

    @pl.when(wid == 0)
    def _():
        pltpu.sync_copy(offs_v, offs_h)
        pltpu.sync_copy(cnts_v, cnts_h)

    base = wid * EPT

    def chunk(c, _):
        eo = base + c * CHS
        pltpu.sync_copy(src_h.at[pl.ds(eo, CHS)], sv)
        pltpu.sync_copy(dst_h.at[pl.ds(eo, CHS)], dv)
        pltpu.sync_copy(ew_h.at[pl.ds(eo, CHS)], ewv)
        gcp = []
        for j in range(8):
            sl = pl.ds(j * 128, 128)
            gcp.append(pltpu.async_copy(dinv_h.at[sv.at[sl]], dsv.at[sl], sem))
            gcp.append(pltpu.async_copy(dinv_h.at[dv.at[sl]], ddv.at[sl], sem))
        for cp in gcp:
            cp.wait()

        def vec(g, _):
            sl = pl.ds(g * 16, 16)
            wv[sl] = -(dsv[sl] * ewv[sl] * ddv[sl])
            return 0
        lax.fori_loop(0, CHS // 16, vec, 0)

        rem = jnp.minimum(EPT - c * CHS, CHS)

        def place(g, _):
            b16 = lax.shift_right_logical(dv[pl.ds(g * 16, 16)], 8)
            pvec = jnp.zeros((16,), jnp.int32)
            for l in range(16):
                bb = b16[l]
                i_abs = g * 16 + l
                is_real = i_abs < rem
                p_real = ctr[bb]
                p = jnp.where(is_real, p_real, SCRATCH + wid * CHS + i_abs)
                ctr[bb] = jnp.where(is_real, p_real + 1, p_real)
                pvec = jnp.where(lane == l, p, pvec)
            r = lax.shift_right_logical(g, 3)
            q = jnp.bitwise_and(g, 7) * 16
            pos2d[r, pl.ds(q, 16)] = pvec
            return 0
        lax.fori_loop(0, CHS // 16, place, 0)

        scp = []
        for j in range(8):
            idx = pos2d.at[j]
            sl = pl.ds(j * 128, 128)
            scp.append(pltpu.async_copy(sv.at[sl], srcs_h.at[idx], sem))
            scp.append(pltpu.async_copy(dv.at[sl], dsts_h.at[idx], sem))
            scp.append(pltpu.async_copy(wv.at[sl], ws_h.at[idx], sem))
        for cp in scp:
            cp.wait()
        return 0
    lax.fori_loop(0, NCHS, chunk, 0)


def _make_prop(D, first):
    GPE = D // 16
    FLATB = NBLK * GPE
    G = 64 if D == 256 else 128
    GSH = 6 if D == 256 else 7

    def body(*args):
        if first:
            (x_h, srcs_h, dsts_h, ws_h, offs_h, cnts_h, y_h,
             acc, rows2, xp, sidx2, wvv2, dvv2, offs_v, cnts_v, offs_sm,
             cnts_sm, sem_e, sem_g) = args
            xprev_h = None
        else:
            (x_h, xprev_h, srcs_h, dsts_h, ws_h, offs_h, cnts_h, y_h,
             acc, rows2, xp, sidx2, wvv2, dvv2, offs_v, cnts_v, offs_sm,
             cnts_sm, sem_e, sem_g) = args
        wid = _wid()
        lane = _lane()
        pltpu.sync_copy(offs_h, offs_v)
        pltpu.sync_copy(cnts_h, cnts_v)
        for sg in range(16):
            o16 = offs_v[pl.ds(sg * 16, 16)]
            c16 = cnts_v[pl.ds(sg * 16, 16)]
            for l in range(16):
                offs_sm[sg * 16 + l] = o16[l]
                cnts_sm[sg * 16 + l] = c16[l]

        def bucket(j, _):
            b = wid + NW * j

            @pl.when(b < NB)
            def _():
                nbase = b * NBLK
                fbase = b * FLATB

                def z(i, _):
                    acc[i] = jnp.zeros((16,), jnp.float32)
                    return 0
                lax.fori_loop(0, FLATB, z, 0)

                off0 = pl.multiple_of(offs_sm[b], 8)
                cnt = cnts_sm[b]
                nch = lax.shift_right_logical(cnt + (G - 1), GSH)

                def start_edata(c, slot):
                    eo = off0 + c * G
                    pltpu.async_copy(srcs_h.at[pl.ds(eo, G)], sidx2.at[slot],
                                     sem_e)
                    pltpu.async_copy(ws_h.at[pl.ds(eo, G)], wvv2.at[slot],
                                     sem_e)
                    pltpu.async_copy(dsts_h.at[pl.ds(eo, G)], dvv2.at[slot],
                                     sem_e)

                def wait_edata(slot):
                    pltpu.make_async_copy(srcs_h.at[pl.ds(0, G)],
                                          sidx2.at[slot], sem_e).wait()
                    pltpu.make_async_copy(ws_h.at[pl.ds(0, G)],
                                          wvv2.at[slot], sem_e).wait()
                    pltpu.make_async_copy(dsts_h.at[pl.ds(0, G)],
                                          dvv2.at[slot], sem_e).wait()

                def start_gather(slot):
                    def cl(g, _):
                        sl = pl.ds(g * 16, 16)
                        sidx2[slot, sl] = jnp.minimum(
                            jnp.maximum(sidx2[slot, sl], 0), N_PAD - 1)
                        return 0
                    lax.fori_loop(0, G // 16, cl, 0)
                    pltpu.async_copy(x_h.at[sidx2.at[slot]],
                                     rows2.at[pl.ds(slot * G, G)], sem_g)

                def wait_gather(slot):
                    pltpu.make_async_copy(x_h.at[pl.ds(0, G)],
                                          rows2.at[pl.ds(slot * G, G)],
                                          sem_g).wait()

                @pl.when(nch > 0)
                def _():
                    start_edata(0, 0)
                    wait_edata(0)
                    start_gather(0)

                    @pl.when(nch > 1)
                    def _():
                        start_edata(1, 1)

                def chunk(c, _):
                    slot = jnp.bitwise_and(c, 1)
                    wait_gather(slot)

                    @pl.when(c + 1 < nch)
                    def _():
                        wait_edata(1 - slot)
                        start_gather(1 - slot)

                    rem = cnt - c * G

                    def group(g, _):
                        sl = pl.ds(g * 16, 16)
                        w16 = jnp.where(lane < rem - g * 16,
                                        wvv2[slot, sl], 0.0)
                        d16 = jnp.minimum(
                            jnp.maximum(dvv2[slot, sl] - nbase, 0),
                            NBLK - 1) * GPE
                        rbase = slot * G + g * 16
                        for l in range(16):
                            fr = d16[l]
                            wi = w16[l]
                            for gg in range(GPE):
                                plsc.addupdate(
                                    acc.at[fr + gg],
                                    wi * rows2[rbase + l, pl.ds(gg * 16, 16)])
                        return 0
                    lax.fori_loop(0, G // 16, group, 0)

                    @pl.when(c + 2 < nch)
                    def _():
                        start_edata(c + 2, slot)
                    return 0
                lax.fori_loop(0, nch, chunk, 0)

                if first:
                    pltpu.sync_copy(acc, y_h.at[pl.ds(fbase, FLATB)])
                else:
                    def comb(q, _):
                        pltpu.sync_copy(
                            xprev_h.at[pl.ds(fbase + q * 512, 512)], xp)

                        def cv(v, _):
                            a = acc[q * 512 + v]
                            acc[q * 512 + v] = 2.0 * a - xp[v]
                            return 0
                        lax.fori_loop(0, 512, cv, 0)
                        return 0
                    lax.fori_loop(0, FLATB // 512, comb, 0)
                    pltpu.sync_copy(acc, y_h.at[pl.ds(fbase, FLATB)])
            return 0
        lax.fori_loop(0, (NB + NW - 1) // NW, bucket, 0)

    return pl.kernel(
        body,
        out_type=jax.ShapeDtypeStruct((N_PAD * GPE, 16), jnp.float32),
        mesh=_MESH,
        compiler_params=_NOTC,
        scratch_types=[pltpu.VMEM((FLATB, 16), jnp.float32),
                       pltpu.VMEM((2 * G, D), jnp.float32),
                       pltpu.VMEM((512, 16), jnp.float32),
                       pltpu.VMEM((2, G), jnp.int32),
                       pltpu.VMEM((2, G), jnp.float32),
                       pltpu.VMEM((2, G), jnp.int32),
                       pltpu.VMEM((256,), jnp.int32),
                       pltpu.VMEM((256,), jnp.int32),
                       pltpu.SMEM((256,), jnp.int32),
                       pltpu.SMEM((256,), jnp.int32),
                       pltpu.SemaphoreType.DMA,
                       pltpu.SemaphoreType.DMA],
    )


_prop64_first = _make_prop(64, True)
_prop64_rec = _make_prop(64, False)
_prop256_first = _make_prop(256, True)
_prop256_rec = _make_prop(256, False)


def _mm_body(relu):
    def body(x0, x1, x2, x3, x4, w_ref, b_ref, o_ref):
        W = w_ref[...]
        acc = jnp.dot(x0[...], W[0], preferred_element_type=jnp.float32)
        acc = acc + jnp.dot(x1[...], W[1], preferred_element_type=jnp.float32)
        acc = acc + jnp.dot(x2[...], W[2], preferred_element_type=jnp.float32)
        acc = acc + jnp.dot(x3[...], W[3], preferred_element_type=jnp.float32)
        acc = acc + jnp.dot(x4[...], W[4], preferred_element_type=jnp.float32)
        acc = acc + b_ref[...]
        if relu:
            acc = jnp.maximum(acc, 0.0)
        o_ref[...] = acc
    return body


def _mm(xs, Wt, b, relu):
    BM = 256
    DI = Wt.shape[1]
    DO = Wt.shape[2]
    in_specs = ([pl.BlockSpec((BM, DI), lambda i: (i, 0)) for _ in range(5)]
                + [pl.BlockSpec((5, DI, DO), lambda i: (0, 0, 0)),
                   pl.BlockSpec((1, DO), lambda i: (0, 0))])
    return pl.pallas_call(
        _mm_body(relu),
        grid=(N_PAD // BM,),
        in_specs=in_specs,
        out_specs=pl.BlockSpec((BM, DO), lambda i: (i, 0)),
        out_shape=jax.ShapeDtypeStruct((N_PAD, DO), jnp.float32),
    )(*xs, Wt, b.reshape(1, -1))


def kernel(x, edge_index, train_edge_weight, W1, b1, W2, b2):
    src = edge_index[0]
    dst = edge_index[1]
    srcp = jnp.pad(src, (0, E_IN - E))
    dstp = jnp.pad(dst, (0, E_IN - E))
    ewp = jnp.pad(train_edge_weight, (0, E_IN - E))
    xp = jnp.pad(x, ((0, N_PAD - N), (0, 0)))

    pdeg, hist = _k_deghist(srcp, dstp, ewp)
    dinv = _k_dinv(pdeg)
    es = _k_sort(srcp, dstp, ewp, dinv, hist)

    def flat(a, d):
        return a.reshape(N_PAD * (d // 16), 16)

    def unflat(a, d):
        return a.reshape(N_PAD, d)

    T0 = xp
    T0f = flat(T0, 64)
    T1f = _prop64_first(T0, *es)
    T1 = unflat(T1f, 64)
    T2f = _prop64_rec(T1, T0f, *es)
    T2 = unflat(T2f, 64)
    T3f = _prop64_rec(T2, T1f, *es)
    T3 = unflat(T3f, 64)
    T4f = _prop64_rec(T3, T2f, *es)
    T4 = unflat(T4f, 64)
    h = _mm([T0, T1, T2, T3, T4], W1, b1, relu=True)

    S0 = h
    S0f = flat(S0, 256)
    S1f = _prop256_first(S0, *es)
    S1 = unflat(S1f, 256)
    S2f = _prop256_rec(S1, S0f, *es)
    S2 = unflat(S2f, 256)
    S3f = _prop256_rec(S2, S1f, *es)
    S3 = unflat(S3f, 256)
    S4f = _prop256_rec(S3, S2f, *es)
    S4 = unflat(S4f, 256)
    out = _mm([S0, S1, S2, S3, S4], W2, b2, relu=False)

    return out[:N]
